# baseline (speedup 1.0000x reference)
import functools
import math

import jax
import jax.numpy as jnp
import numpy as np
from jax import lax
from jax.experimental import pallas as pl
from jax.experimental.pallas import tpu as pltpu
from jax.experimental.pallas import tpu_sc as plsc

N = 10000
E = 160000
D = 256
H = 8
DK = 32
HD = D // 2
NB = 10
BR = N // NB

NTILES = 16
EP = E // NTILES
C = 16
NCH = EP // C
NG = C // 16
NP = 10240
RT = NP // NTILES


def _qkv_body(x_ref, wq_ref, wk_ref, wv_ref, b_ref, q_ref, k_ref, v_ref):
    xb = x_ref[...]
    q_ref[...] = (
        jnp.dot(xb, wq_ref[...], preferred_element_type=jnp.float32)
        + b_ref[0:1, 0:HD]
    )
    k_ref[...] = (
        jnp.dot(xb, wk_ref[...], preferred_element_type=jnp.float32)
        + b_ref[0:1, HD : 2 * HD]
    )
    v_ref[...] = (
        jnp.dot(xb, wv_ref[...], preferred_element_type=jnp.float32)
        + b_ref[0:1, 2 * HD : 3 * HD]
    )


def _qkv_proj(x, wq_t, wk_t, wv_t, b3):
    sds = jax.ShapeDtypeStruct((2 * N, HD), jnp.float32)
    return pl.pallas_call(
        _qkv_body,
        out_shape=(sds, sds, sds),
        grid=(2, NB),
        in_specs=[
            pl.BlockSpec((BR, D), lambda c, i: (i, 0)),
            pl.BlockSpec((D, HD), lambda c, i: (0, c)),
            pl.BlockSpec((D, HD), lambda c, i: (0, c)),
            pl.BlockSpec((D, HD), lambda c, i: (0, c)),
            pl.BlockSpec((1, 3 * HD), lambda c, i: (0, c)),
        ],
        out_specs=(
            pl.BlockSpec((BR, HD), lambda c, i: (c * NB + i, 0)),
            pl.BlockSpec((BR, HD), lambda c, i: (c * NB + i, 0)),
            pl.BlockSpec((BR, HD), lambda c, i: (c * NB + i, 0)),
        ),
    )(x, wq_t, wk_t, wv_t, b3)


ND = NP // 8
DT = ND // NTILES


def _sc_body(
    q2, k2, v2, rows_h, cols_h, biast,
    y_out, dpk_out,
    rows_v, cols_v, qidx_v, cidx_v, didx_v, abuf, bbuf, vbuf, biasb,
    yacc, semq, semk, semv,
):
    c = lax.axis_index("c")
    s = lax.axis_index("s")
    coff = c * N
    aoff = c * NP
    c4 = c * 4

    zv = jnp.zeros((16,), jnp.float32)
    for j in range(C):
        for kk in range(HD // 16):
            abuf[j, pl.ds(kk * 16, 16)] = zv
    for m in range(RT // C):
        pltpu.sync_copy(abuf, yacc.at[pl.ds(s * RT + m * C, C)])
    for m in range(DT // C):
        pltpu.sync_copy(abuf, yacc.at[pl.ds(NP + s * DT + m * C, C)])
    plsc.subcore_barrier()

    base0 = s * EP
    lane = lax.iota(jnp.int32, 16)

    def chunk_body(t, carry):
        base = base0 + t * C
        pltpu.sync_copy(rows_h.at[pl.ds(base, C)], rows_v)
        pltpu.sync_copy(cols_h.at[pl.ds(base, C)], cols_v)
        for hl in range(4):
            pltpu.sync_copy(
                biast.at[pl.ds((c4 + hl) * E + base, C)],
                biasb.at[pl.ds(hl * C, C)],
            )
        for j in range(C // 16):
            sl = pl.ds(j * 16, 16)
            qidx_v[sl] = rows_v[sl] + coff
            cidx_v[sl] = cols_v[sl] + coff
            didx_v[sl] = jnp.right_shift(rows_v[sl], 3) + NP
        cq = pltpu.async_copy(q2.at[qidx_v], abuf, semq)
        ck = pltpu.async_copy(k2.at[cidx_v], bbuf, semk)
        cv = pltpu.async_copy(v2.at[cidx_v], vbuf, semv)
        cq.wait()
        ck.wait()
        cv.wait()

        for g in range(NG):
            le = lane + g * 16
            rows16 = rows_v[pl.ds(g * 16, 16)]
            cbase = jnp.left_shift(jnp.bitwise_and(rows16, 7), 4)
            accs = [jnp.zeros((16,), jnp.float32) for _ in range(4)]
            for d in range(HD):
                fd = jnp.full((16,), d, jnp.int32)
                qd = plsc.load_gather(abuf, [le, fd])
                kd = plsc.load_gather(bbuf, [le, fd])
                accs[d // DK] = accs[d // DK] + qd * kd
            svals = []
            for hl in range(4):
                bt = biasb[pl.ds(hl * C + g * 16, 16)]
                svals.append(jnp.exp(accs[hl] + bt))
            for j in range(16):
                for kk in range(HD // 16):
                    abuf[g * 16 + j, pl.ds(kk * 16, 16)] = zv
            for hl in range(4):
                plsc.store_scatter(abuf, [le, cbase + hl], svals[hl])
            for d in range(HD):
                fd = jnp.full((16,), d, jnp.int32)
                vd = plsc.load_gather(vbuf, [le, fd])
                plsc.store_scatter(vbuf, [le, fd], svals[d // DK] * vd)

        pltpu.sync_copy(vbuf, yacc.at[rows_v], add=True)
        pltpu.sync_copy(abuf, yacc.at[didx_v], add=True)
        return carry

    lax.fori_loop(0, NCH, chunk_body, 0)
    plsc.subcore_barrier()

    for m in range(RT // C):
        sl = pl.ds(s * RT + m * C, C)
        pltpu.sync_copy(yacc.at[sl], abuf)
        pltpu.sync_copy(abuf, y_out.at[pl.ds(aoff + s * RT + m * C, C)])
    for m in range(DT // C):
        pltpu.sync_copy(yacc.at[pl.ds(NP + s * DT + m * C, C)], abuf)
        pltpu.sync_copy(abuf, dpk_out.at[pl.ds(c * ND + s * DT + m * C, C)])


def _sc_call():
  return functools.partial(
    pl.kernel,
    _sc_body,
    out_type=(
        jax.ShapeDtypeStruct((2 * NP, HD), jnp.float32),
        jax.ShapeDtypeStruct((2 * ND, HD), jnp.float32),
    ),
    mesh=plsc.VectorSubcoreMesh(core_axis_name="c", subcore_axis_name="s"),
    compiler_params=pltpu.CompilerParams(needs_layout_passes=False),
    scratch_types=(
        pltpu.VMEM((C,), jnp.int32),
        pltpu.VMEM((C,), jnp.int32),
        pltpu.VMEM((C,), jnp.int32),
        pltpu.VMEM((C,), jnp.int32),
        pltpu.VMEM((C,), jnp.int32),
        pltpu.VMEM((C, HD), jnp.float32),
        pltpu.VMEM((C, HD), jnp.float32),
        pltpu.VMEM((C, HD), jnp.float32),
        pltpu.VMEM((4 * C,), jnp.float32),
        pltpu.VMEM_SHARED((NP + ND, HD), jnp.float32),
        pltpu.SemaphoreType.DMA,
        pltpu.SemaphoreType.DMA,
        pltpu.SemaphoreType.DMA,
    ),
  )()


def _out_body(y0_ref, y1_ref, d0_ref, d1_ref, r16_ref, wo_ref, bo_ref, o_ref):
    dinv0 = 1.0 / jnp.where(d0_ref[...] == 0.0, 1.0, d0_ref[...])
    dinv1 = 1.0 / jnp.where(d1_ref[...] == 0.0, 1.0, d1_ref[...])
    rep0 = jnp.dot(dinv0, r16_ref[...], preferred_element_type=jnp.float32)
    rep1 = jnp.dot(dinv1, r16_ref[...], preferred_element_type=jnp.float32)
    o_ref[...] = (
        jnp.dot(y0_ref[...] * rep0, wo_ref[0:HD, :],
                preferred_element_type=jnp.float32)
        + jnp.dot(y1_ref[...] * rep1, wo_ref[HD : 2 * HD, :],
                  preferred_element_type=jnp.float32)
        + bo_ref[...]
    )


def _out_proj(y0, y1, d0, d1, r16, wo_t, bo):
    return pl.pallas_call(
        _out_body,
        out_shape=jax.ShapeDtypeStruct((N, D), jnp.float32),
        grid=(NB,),
        in_specs=[
            pl.BlockSpec((BR, HD), lambda i: (i, 0)),
            pl.BlockSpec((BR, HD), lambda i: (i, 0)),
            pl.BlockSpec((BR, 16), lambda i: (i, 0)),
            pl.BlockSpec((BR, 16), lambda i: (i, 0)),
            pl.BlockSpec((16, HD), lambda i: (0, 0)),
            pl.BlockSpec((D, D), lambda i: (0, 0)),
            pl.BlockSpec((1, D), lambda i: (0, 0)),
        ],
        out_specs=pl.BlockSpec((BR, D), lambda i: (i, 0)),
    )(y0, y1, d0, d1, r16, wo_t, bo.reshape(1, D))


_R16 = np.zeros((16, HD), np.float32)
for _h in range(4):
    _R16[_h, _h * DK : (_h + 1) * DK] = 1.0


def kernel(x, edge_index, att_bias, Wq, bq, Wk, bk, Wv, bv, Wo, bo):
    scale = 1.0 / math.sqrt(DK)
    b3 = (
        jnp.concatenate([bq * scale, bk, bv])
        .reshape(3, 2, HD)
        .transpose(1, 0, 2)
        .reshape(1, 2 * 3 * HD)
    )
    q2, k2, v2 = _qkv_proj(x, Wq.T * scale, Wk.T, Wv.T, b3)

    rows = edge_index[0]
    cols = edge_index[1]
    biast = att_bias.T.reshape(H * E)
    y2, dpk = _sc_call()(q2, k2, v2, rows, cols, biast)

    d2a = dpk[:ND].reshape(NP, 16)
    d2b = dpk[ND:].reshape(NP, 16)
    r16 = jnp.asarray(_R16)
    return _out_proj(y2[:N], y2[NP : NP + N], d2a[:N], d2b[:N], r16, Wo.T, bo)


# speedup vs baseline: 1.1221x; 1.1221x over previous
import functools
import math

import jax
import jax.numpy as jnp
import numpy as np
from jax import lax
from jax.experimental import pallas as pl
from jax.experimental.pallas import tpu as pltpu
from jax.experimental.pallas import tpu_sc as plsc

N = 10000
E = 160000
D = 256
H = 8
DK = 32
HD = D // 2
NB = 10
BR = N // NB

NTILES = 16
EP = E // NTILES
C = 16
NCH = EP // C
NG = C // 16
NP = 10240
RT = NP // NTILES


def _qkv_body(x_ref, wq_ref, wk_ref, wv_ref, b_ref, q_ref, k_ref, v_ref):
    xb = x_ref[...]
    q_ref[...] = (
        jnp.dot(xb, wq_ref[...], preferred_element_type=jnp.float32)
        + b_ref[0:1, 0:HD]
    )
    k_ref[...] = (
        jnp.dot(xb, wk_ref[...], preferred_element_type=jnp.float32)
        + b_ref[0:1, HD : 2 * HD]
    )
    v_ref[...] = (
        jnp.dot(xb, wv_ref[...], preferred_element_type=jnp.float32)
        + b_ref[0:1, 2 * HD : 3 * HD]
    )


def _qkv_proj(x, wq_t, wk_t, wv_t, b3):
    sds = jax.ShapeDtypeStruct((2 * N, HD), jnp.float32)
    return pl.pallas_call(
        _qkv_body,
        out_shape=(sds, sds, sds),
        grid=(2, NB),
        in_specs=[
            pl.BlockSpec((BR, D), lambda c, i: (i, 0)),
            pl.BlockSpec((D, HD), lambda c, i: (0, c)),
            pl.BlockSpec((D, HD), lambda c, i: (0, c)),
            pl.BlockSpec((D, HD), lambda c, i: (0, c)),
            pl.BlockSpec((1, 3 * HD), lambda c, i: (0, c)),
        ],
        out_specs=(
            pl.BlockSpec((BR, HD), lambda c, i: (c * NB + i, 0)),
            pl.BlockSpec((BR, HD), lambda c, i: (c * NB + i, 0)),
            pl.BlockSpec((BR, HD), lambda c, i: (c * NB + i, 0)),
        ),
    )(x, wq_t, wk_t, wv_t, b3)


ND = NP // 8
DT = ND // NTILES


def _sc_body(
    q2, k2, v2, rows_h, cols_h, biast,
    y_out, dpk_out,
    rows_v, cols_v, qidx_v, cidx_v, didx_v, abuf, bbuf, vbuf, biasb,
    yacc, semq, semk, semv,
):
    c = lax.axis_index("c")
    s = lax.axis_index("s")
    coff = c * N
    aoff = c * NP
    c4 = c * 4

    zv = jnp.zeros((16,), jnp.float32)
    for j in range(C):
        for kk in range(HD // 16):
            abuf[j, pl.ds(kk * 16, 16)] = zv
    for m in range(RT // C):
        pltpu.sync_copy(abuf, yacc.at[pl.ds(s * RT + m * C, C)])
    for m in range(DT // C):
        pltpu.sync_copy(abuf, yacc.at[pl.ds(NP + s * DT + m * C, C)])
    plsc.subcore_barrier()

    base0 = s * EP
    lane = lax.iota(jnp.int32, 16)

    def chunk_body(t, carry):
        base = base0 + t * C
        pltpu.sync_copy(rows_h.at[pl.ds(base, C)], rows_v)
        pltpu.sync_copy(cols_h.at[pl.ds(base, C)], cols_v)
        pltpu.sync_copy(biast.at[pl.ds(c * (4 * E) + base * 4, 4 * C)], biasb)
        for j in range(C // 16):
            sl = pl.ds(j * 16, 16)
            qidx_v[sl] = rows_v[sl] + coff
            cidx_v[sl] = cols_v[sl] + coff
            didx_v[sl] = jnp.right_shift(rows_v[sl], 3) + NP
        cq = pltpu.async_copy(q2.at[qidx_v], abuf, semq)
        ck = pltpu.async_copy(k2.at[cidx_v], bbuf, semk)
        cv = pltpu.async_copy(v2.at[cidx_v], vbuf, semv)
        cq.wait()
        ck.wait()
        cv.wait()

        for g in range(NG):
            le = lane + g * 16
            rows16 = rows_v[pl.ds(g * 16, 16)]
            cbase = jnp.left_shift(jnp.bitwise_and(rows16, 7), 4)
            accs = [jnp.zeros((16,), jnp.float32) for _ in range(4)]
            for d in range(HD):
                fd = jnp.full((16,), d, jnp.int32)
                qd = plsc.load_gather(abuf, [le, fd])
                kd = plsc.load_gather(bbuf, [le, fd])
                accs[d // DK] = accs[d // DK] + qd * kd
            svals = []
            for hl in range(4):
                bt = biasb[pl.ds(hl * C + g * 16, 16)]
                svals.append(jnp.exp(accs[hl] + bt))
            for j in range(16):
                for kk in range(HD // 16):
                    abuf[g * 16 + j, pl.ds(kk * 16, 16)] = zv
            for hl in range(4):
                plsc.store_scatter(abuf, [le, cbase + hl], svals[hl])
            for d in range(HD):
                fd = jnp.full((16,), d, jnp.int32)
                vd = plsc.load_gather(vbuf, [le, fd])
                plsc.store_scatter(vbuf, [le, fd], svals[d // DK] * vd)

        pltpu.sync_copy(vbuf, yacc.at[rows_v], add=True)
        pltpu.sync_copy(abuf, yacc.at[didx_v], add=True)
        return carry

    lax.fori_loop(0, NCH, chunk_body, 0)
    plsc.subcore_barrier()

    for m in range(RT // C):
        sl = pl.ds(s * RT + m * C, C)
        pltpu.sync_copy(yacc.at[sl], abuf)
        pltpu.sync_copy(abuf, y_out.at[pl.ds(aoff + s * RT + m * C, C)])
    for m in range(DT // C):
        pltpu.sync_copy(yacc.at[pl.ds(NP + s * DT + m * C, C)], abuf)
        pltpu.sync_copy(abuf, dpk_out.at[pl.ds(c * ND + s * DT + m * C, C)])


def _sc_call():
  return functools.partial(
    pl.kernel,
    _sc_body,
    out_type=(
        jax.ShapeDtypeStruct((2 * NP, HD), jnp.float32),
        jax.ShapeDtypeStruct((2 * ND, HD), jnp.float32),
    ),
    mesh=plsc.VectorSubcoreMesh(core_axis_name="c", subcore_axis_name="s"),
    compiler_params=pltpu.CompilerParams(needs_layout_passes=False),
    scratch_types=(
        pltpu.VMEM((C,), jnp.int32),
        pltpu.VMEM((C,), jnp.int32),
        pltpu.VMEM((C,), jnp.int32),
        pltpu.VMEM((C,), jnp.int32),
        pltpu.VMEM((C,), jnp.int32),
        pltpu.VMEM((C, HD), jnp.float32),
        pltpu.VMEM((C, HD), jnp.float32),
        pltpu.VMEM((C, HD), jnp.float32),
        pltpu.VMEM((4 * C,), jnp.float32),
        pltpu.VMEM_SHARED((NP + ND, HD), jnp.float32),
        pltpu.SemaphoreType.DMA,
        pltpu.SemaphoreType.DMA,
        pltpu.SemaphoreType.DMA,
    ),
  )()


def _out_body(y0_ref, y1_ref, d0_ref, d1_ref, r16_ref, wo_ref, bo_ref, o_ref):
    dinv0 = 1.0 / jnp.where(d0_ref[...] == 0.0, 1.0, d0_ref[...])
    dinv1 = 1.0 / jnp.where(d1_ref[...] == 0.0, 1.0, d1_ref[...])
    rep0 = jnp.dot(dinv0, r16_ref[...], preferred_element_type=jnp.float32)
    rep1 = jnp.dot(dinv1, r16_ref[...], preferred_element_type=jnp.float32)
    o_ref[...] = (
        jnp.dot(y0_ref[...] * rep0, wo_ref[0:HD, :],
                preferred_element_type=jnp.float32)
        + jnp.dot(y1_ref[...] * rep1, wo_ref[HD : 2 * HD, :],
                  preferred_element_type=jnp.float32)
        + bo_ref[...]
    )


def _out_proj(y0, y1, d0, d1, r16, wo_t, bo):
    return pl.pallas_call(
        _out_body,
        out_shape=jax.ShapeDtypeStruct((N, D), jnp.float32),
        grid=(NB,),
        in_specs=[
            pl.BlockSpec((BR, HD), lambda i: (i, 0)),
            pl.BlockSpec((BR, HD), lambda i: (i, 0)),
            pl.BlockSpec((BR, 16), lambda i: (i, 0)),
            pl.BlockSpec((BR, 16), lambda i: (i, 0)),
            pl.BlockSpec((16, HD), lambda i: (0, 0)),
            pl.BlockSpec((D, D), lambda i: (0, 0)),
            pl.BlockSpec((1, D), lambda i: (0, 0)),
        ],
        out_specs=pl.BlockSpec((BR, D), lambda i: (i, 0)),
    )(y0, y1, d0, d1, r16, wo_t, bo.reshape(1, D))


_R16 = np.zeros((16, HD), np.float32)
for _h in range(4):
    _R16[_h, _h * DK : (_h + 1) * DK] = 1.0


def kernel(x, edge_index, att_bias, Wq, bq, Wk, bk, Wv, bv, Wo, bo):
    scale = 1.0 / math.sqrt(DK)
    b3 = (
        jnp.concatenate([bq * scale, bk, bv])
        .reshape(3, 2, HD)
        .transpose(1, 0, 2)
        .reshape(1, 2 * 3 * HD)
    )
    q2, k2, v2 = _qkv_proj(x, Wq.T * scale, Wk.T, Wv.T, b3)

    rows = edge_index[0]
    cols = edge_index[1]
    biast = (
        att_bias.reshape(E // C, C, 2, 4).transpose(2, 0, 3, 1).reshape(H * E)
    )
    y2, dpk = _sc_call()(q2, k2, v2, rows, cols, biast)

    d2a = dpk[:ND].reshape(NP, 16)
    d2b = dpk[ND:].reshape(NP, 16)
    r16 = jnp.asarray(_R16)
    return _out_proj(y2[:N], y2[NP : NP + N], d2a[:N], d2b[:N], r16, Wo.T, bo)
